# Initial kernel scaffold; baseline (speedup 1.0000x reference)
import jax
import jax.numpy as jnp
from jax.experimental import pallas as pl


def kernel(goldstandard, emb_table, W_out, b_out, W_val, b_val):
    raise NotImplementedError("write your pallas kernel here")


# speedup vs baseline: 2.9419x; 2.9419x over previous
import functools

import jax
import jax.numpy as jnp
from jax import lax
from jax.experimental import pallas as pl
from jax.experimental.pallas import tpu as pltpu
from jax.experimental.pallas import tpu_sc as plsc

_B, _L, _E, _OV = 16384, 200, 64, 256

_NW = 32
_BPW = _B // _NW
_CHUNK = 2
_T = _BPW // _CHUNK
_SUBLEN = 100
_NSUB = (_CHUNK * _L) // _SUBLEN
_GROUPS = _E // 16
_UNROLL = 8


def _embed_sum_sc(idx3, emb_table):
    mesh = plsc.VectorSubcoreMesh(core_axis_name="c", subcore_axis_name="s")

    @functools.partial(
        pl.kernel,
        out_type=jax.ShapeDtypeStruct((_B, _E), jnp.float32),
        mesh=mesh,
        scratch_types=[
            pltpu.VMEM((2, _NSUB, _SUBLEN), jnp.int32),
            pltpu.VMEM((2, _CHUNK * _L, _E), jnp.float32),
            pltpu.VMEM((_BPW, _E), jnp.float32),
            pltpu.SemaphoreType.DMA,
            pltpu.SemaphoreType.DMA,
        ],
        compiler_params=pltpu.CompilerParams(use_tc_tiling_on_sc=False),
    )
    def k(idx_hbm, table_hbm, out_hbm, idx_v, rows_v, out_v, sem0, sem1):
        wid = lax.axis_index("s") * 2 + lax.axis_index("c")
        row0 = wid * _T

        def fire(t, buf, sem):
            pltpu.sync_copy(idx_hbm.at[row0 + t], idx_v.at[buf])
            for j in range(_NSUB):
                pltpu.async_copy(
                    table_hbm.at[idx_v.at[buf, j]],
                    rows_v.at[buf, pl.ds(j * _SUBLEN, _SUBLEN)],
                    sem,
                )

        def drain(buf, sem):
            for j in range(_NSUB):
                pltpu.make_async_copy(
                    table_hbm.at[idx_v.at[buf, j]],
                    rows_v.at[buf, pl.ds(j * _SUBLEN, _SUBLEN)],
                    sem,
                ).wait()

        def accum(t, buf):
            for c in range(_CHUNK):
                base = c * _L

                def body(jj, accs):
                    j = base + jj * _UNROLL
                    new = []
                    for g in range(_GROUPS):
                        a = accs[g]
                        for u in range(_UNROLL):
                            a = a + rows_v[buf, j + u, pl.ds(g * 16, 16)]
                        new.append(a)
                    return tuple(new)

                zero = jnp.zeros((16,), jnp.float32)
                accs = lax.fori_loop(0, _L // _UNROLL, body, (zero,) * _GROUPS)
                for g in range(_GROUPS):
                    out_v[t * _CHUNK + c, pl.ds(g * 16, 16)] = accs[g]

        fire(0, 0, sem0)

        def pair(i, carry):
            t0 = 2 * i
            fire(t0 + 1, 1, sem1)
            drain(0, sem0)
            accum(t0, 0)

            @pl.when(t0 + 2 < _T)
            def _():
                fire(t0 + 2, 0, sem0)

            drain(1, sem1)
            accum(t0 + 1, 1)
            return carry

        lax.fori_loop(0, _T // 2, pair, 0)
        pltpu.sync_copy(out_v, out_hbm.at[pl.ds(wid * _BPW, _BPW)])

    return k(idx3, emb_table)


def _heads_tc(hid, w_outT, b_out2, w_val2, b_val2, u):
    blk = 512
    grid = _B // blk

    def body(hid_ref, w_ref, b_ref, wv_ref, bv_ref, u_ref,
             comms_ref, probs_ref, val_ref):
        h = hid_ref[...]
        logits = jnp.dot(h, w_ref[...], preferred_element_type=jnp.float32)
        logits = logits + b_ref[...]
        p = jax.nn.sigmoid(logits)
        probs_ref[...] = p
        comms_ref[...] = (u_ref[...] < p).astype(jnp.float32)
        v = jnp.sum(h * wv_ref[...], axis=1, keepdims=True) + bv_ref[0, 0]
        val_ref[...] = v

    return pl.pallas_call(
        body,
        grid=(grid,),
        in_specs=[
            pl.BlockSpec((blk, _E), lambda i: (i, 0)),
            pl.BlockSpec((_E, _OV), lambda i: (0, 0)),
            pl.BlockSpec((1, _OV), lambda i: (0, 0)),
            pl.BlockSpec((1, _E), lambda i: (0, 0)),
            pl.BlockSpec(memory_space=pltpu.SMEM),
            pl.BlockSpec((blk, _OV), lambda i: (i, 0)),
        ],
        out_specs=[
            pl.BlockSpec((blk, _OV), lambda i: (i, 0)),
            pl.BlockSpec((blk, _OV), lambda i: (i, 0)),
            pl.BlockSpec((blk, 1), lambda i: (i, 0)),
        ],
        out_shape=[
            jax.ShapeDtypeStruct((_B, _OV), jnp.float32),
            jax.ShapeDtypeStruct((_B, _OV), jnp.float32),
            jax.ShapeDtypeStruct((_B, 1), jnp.float32),
        ],
    )(hid, w_outT, b_out2, w_val2, b_val2, u)


def kernel(goldstandard, emb_table, W_out, b_out, W_val, b_val):
    idx3 = goldstandard.reshape(_B // _CHUNK, _NSUB, _SUBLEN)
    hid = _embed_sum_sc(idx3, emb_table)
    u = jax.random.uniform(jax.random.key(1), (_B, _OV), jnp.float32)
    comms, probs, value = _heads_tc(
        hid,
        W_out.T,
        b_out.reshape(1, _OV),
        W_val,
        b_val.reshape(1, 1),
        u,
    )
    return comms, probs, value
